# Initial kernel scaffold; baseline (speedup 1.0000x reference)
import jax
import jax.numpy as jnp
from jax.experimental import pallas as pl


def kernel(inputs, table):
    raise NotImplementedError("write your pallas kernel here")


# speedup vs baseline: 1.0600x; 1.0600x over previous
import functools

import jax
import jax.numpy as jnp
from jax import lax
from jax.experimental import pallas as pl
from jax.experimental.pallas import tpu as pltpu
from jax.experimental.pallas import tpu_sc as plsc

N_CLASSES = 1000000
EMBED_DIM = 32
BATCH = 16384
FIELDS = 100

B_TOTAL = BATCH * FIELDS
NUM_CORES = 2
NUM_SUBCORES = 16
NW = NUM_CORES * NUM_SUBCORES
BPW = B_TOTAL // NW
CHUNK = 1024
NCHUNK = BPW // CHUNK
LANES = 16

_mesh = plsc.VectorSubcoreMesh(core_axis_name="c", subcore_axis_name="s")


@functools.partial(
    pl.kernel,
    mesh=_mesh,
    out_type=jax.ShapeDtypeStruct((B_TOTAL, EMBED_DIM), jnp.float32),
    scratch_types=[
        pltpu.VMEM((CHUNK,), jnp.int32),
        pltpu.VMEM((CHUNK, EMBED_DIM), jnp.float32),
        pltpu.SemaphoreType.DMA,
    ],
    compiler_params=pltpu.CompilerParams(use_tc_tiling_on_sc=False),
)
def _emb_lookup(idx_hbm, table_hbm, out_hbm, idx_v, rows_v, sem):
    wid = lax.axis_index("s") * NUM_CORES + lax.axis_index("c")
    base = wid * BPW

    def chunk_body(g, carry):
        off = base + g * CHUNK
        pltpu.sync_copy(idx_hbm.at[pl.ds(off, CHUNK)], idx_v)
        pltpu.async_copy(table_hbm.at[idx_v], rows_v, sem).wait()

        def row_body(i, c):
            a = rows_v[i, pl.ds(0, LANES)]
            b = rows_v[i, pl.ds(LANES, LANES)]
            rows_v[i, pl.ds(0, LANES)] = jnp.abs(a)
            rows_v[i, pl.ds(LANES, LANES)] = jnp.abs(b)
            return c

        lax.fori_loop(0, CHUNK, row_body, 0)
        pltpu.sync_copy(rows_v, out_hbm.at[pl.ds(off, CHUNK)])
        return carry

    lax.fori_loop(0, NCHUNK, chunk_body, 0)


def kernel(inputs, table):
    idx = inputs.reshape(-1).astype(jnp.int32)
    out = _emb_lookup(idx, table)
    return out.reshape(inputs.shape + (EMBED_DIM,))


# speedup vs baseline: 3.1682x; 2.9887x over previous
import functools

import jax
import jax.numpy as jnp
from jax import lax
from jax.experimental import pallas as pl
from jax.experimental.pallas import tpu as pltpu
from jax.experimental.pallas import tpu_sc as plsc

N_CLASSES = 1000000
EMBED_DIM = 32
BATCH = 16384
FIELDS = 100

NUM_CORES = 2
NUM_SUBCORES = 16
NW = NUM_CORES * NUM_SUBCORES
BPW = BATCH // NW
LANES = 16
ETILES = EMBED_DIM // 8
BTILES = BPW // 128

_mesh = plsc.VectorSubcoreMesh(core_axis_name="c", subcore_axis_name="s")


@functools.partial(
    pl.kernel,
    mesh=_mesh,
    out_type=jax.ShapeDtypeStruct(
        (FIELDS, ETILES, BATCH // 128, 8, 128), jnp.float32),
    scratch_types=[
        pltpu.VMEM((BPW,), jnp.int32),
        pltpu.VMEM((BPW, EMBED_DIM), jnp.float32),
        pltpu.VMEM((BTILES, ETILES, 8, 128), jnp.float32),
        pltpu.SemaphoreType.DMA,
    ],
    compiler_params=pltpu.CompilerParams(
        use_tc_tiling_on_sc=False, needs_layout_passes=False),
)
def _emb_lookup(idx_hbm, table_hbm, out_hbm, idx_v, rows_v, t_v, sem):
    wid = lax.axis_index("s") * NUM_CORES + lax.axis_index("c")
    base_b = wid * BPW
    ar16 = jnp.arange(LANES, dtype=jnp.int32)

    def f_body(f, carry):
        pltpu.sync_copy(idx_hbm.at[pl.ds(f * BATCH + base_b, BPW)], idx_v)
        pltpu.async_copy(table_hbm.at[idx_v], rows_v, sem).wait()

        def g_body(g, c):
            q = g // 8
            j = g % 8
            row_ids = g * LANES + ar16
            for e in range(EMBED_DIM):
                vals = plsc.load_gather(
                    rows_v, [row_ids, jnp.full((LANES,), e, jnp.int32)])
                t_v[q, e // 8, e % 8, pl.ds(j * LANES, LANES)] = jnp.abs(vals)
            return c

        lax.fori_loop(0, BPW // LANES, g_body, 0)

        def q_body(q, c):
            for e_t in range(ETILES):
                pltpu.sync_copy(t_v.at[q, e_t],
                                out_hbm.at[f, e_t, wid * BTILES + q])
            return c

        lax.fori_loop(0, BTILES, q_body, 0)
        return carry

    lax.fori_loop(0, FIELDS, f_body, 0)


def kernel(inputs, table):
    idx_t = inputs.T.reshape(-1).astype(jnp.int32)
    out5 = _emb_lookup(idx_t, table)
    return out5.transpose((2, 4, 0, 1, 3)).reshape(BATCH, FIELDS, EMBED_DIM)


# speedup vs baseline: 3.7676x; 1.1892x over previous
import functools

import jax
import jax.numpy as jnp
from jax import lax
from jax.experimental import pallas as pl
from jax.experimental.pallas import tpu as pltpu
from jax.experimental.pallas import tpu_sc as plsc

N_CLASSES = 1000000
EMBED_DIM = 32
BATCH = 16384
FIELDS = 100

NUM_CORES = 2
NUM_SUBCORES = 16
NW = NUM_CORES * NUM_SUBCORES
BPW = BATCH // NW
LANES = 16
ETILES = EMBED_DIM // 8
BTILES = BPW // 128
NPAIR = FIELDS // 2

_mesh = plsc.VectorSubcoreMesh(core_axis_name="c", subcore_axis_name="s")


@functools.partial(
    pl.kernel,
    mesh=_mesh,
    out_type=jax.ShapeDtypeStruct(
        (FIELDS, ETILES, BATCH // 128, 8, 128), jnp.float32),
    scratch_types=[
        pltpu.VMEM((BPW,), jnp.int32),
        pltpu.VMEM((BPW,), jnp.int32),
        pltpu.VMEM((BPW, EMBED_DIM), jnp.float32),
        pltpu.VMEM((BPW, EMBED_DIM), jnp.float32),
        pltpu.VMEM((ETILES, BTILES, 8, 128), jnp.float32),
        pltpu.VMEM((ETILES, BTILES, 8, 128), jnp.float32),
        pltpu.SemaphoreType.DMA,
        pltpu.SemaphoreType.DMA,
        pltpu.SemaphoreType.DMA,
    ],
    compiler_params=pltpu.CompilerParams(
        use_tc_tiling_on_sc=False, needs_layout_passes=False),
)
def _emb_lookup(idx_hbm, table_hbm, out_hbm,
                idx_a, idx_b, rows_a, rows_b, t_a, t_b,
                sem_i, sem_g, sem_o):
    wid = lax.axis_index("s") * NUM_CORES + lax.axis_index("c")
    base_b = wid * BPW
    bt0 = wid * BTILES
    ar16 = jnp.arange(LANES, dtype=jnp.int32)

    def idx_src(f):
        return idx_hbm.at[pl.ds(f * BATCH + base_b, BPW)]

    def transpose_into(rows_v, t_v):
        def g_body(g, c):
            q = g // 8
            j = g % 8
            row_ids = g * LANES + ar16
            for e in range(EMBED_DIM):
                vals = plsc.load_gather(
                    rows_v, [row_ids, jnp.full((LANES,), e, jnp.int32)])
                t_v[e // 8, q, e % 8, pl.ds(j * LANES, LANES)] = jnp.abs(vals)
            return c
        lax.fori_loop(0, BPW // LANES, g_body, 0)

    def fire_out(t_v, f):
        for e_t in range(ETILES):
            pltpu.async_copy(t_v.at[e_t],
                             out_hbm.at[f, e_t, pl.ds(bt0, BTILES)], sem_o)

    def drain_out():
        for e_t in range(ETILES):
            pltpu.make_async_copy(
                t_a.at[e_t], out_hbm.at[0, e_t, pl.ds(bt0, BTILES)],
                sem_o).wait()

    def drain_gather(rows_v):
        pltpu.make_async_copy(
            table_hbm.at[pl.ds(0, BPW)], rows_v, sem_g).wait()

    def drain_idx(idx_v):
        pltpu.make_async_copy(idx_src(0), idx_v, sem_i).wait()

    pltpu.sync_copy(idx_src(0), idx_a)
    pltpu.async_copy(table_hbm.at[idx_a], rows_a, sem_g)
    pltpu.async_copy(idx_src(1), idx_b, sem_i)

    def pair_body(k, carry):
        f0 = 2 * k
        f1 = f0 + 1

        drain_gather(rows_a)

        drain_idx(idx_b)
        pltpu.async_copy(table_hbm.at[idx_b], rows_b, sem_g)

        @pl.when(f0 + 2 < FIELDS)
        def _():
            pltpu.async_copy(idx_src(f0 + 2), idx_a, sem_i)

        @pl.when(k >= 1)
        def _():
            drain_out()

        transpose_into(rows_a, t_a)
        fire_out(t_a, f0)

        drain_gather(rows_b)

        @pl.when(f1 + 1 < FIELDS)
        def _():
            drain_idx(idx_a)
            pltpu.async_copy(table_hbm.at[idx_a], rows_a, sem_g)

        @pl.when(f1 + 2 < FIELDS)
        def _():
            pltpu.async_copy(idx_src(f1 + 2), idx_b, sem_i)

        @pl.when(k >= 1)
        def _():
            drain_out()

        transpose_into(rows_b, t_b)
        fire_out(t_b, f1)
        return carry

    lax.fori_loop(0, NPAIR, pair_body, 0)

    drain_out()
    drain_out()


def kernel(inputs, table):
    idx_t = inputs.T.reshape(-1).astype(jnp.int32)
    out5 = _emb_lookup(idx_t, table)
    return out5.transpose((2, 4, 0, 1, 3)).reshape(BATCH, FIELDS, EMBED_DIM)


# speedup vs baseline: 5.5567x; 1.4749x over previous
import functools

import jax
import jax.numpy as jnp
from jax import lax
from jax.experimental import pallas as pl
from jax.experimental.pallas import tpu as pltpu
from jax.experimental.pallas import tpu_sc as plsc

N_CLASSES = 1000000
EMBED_DIM = 32
BATCH = 16384
FIELDS = 100

NUM_CORES = 2
NUM_SUBCORES = 16
NW = NUM_CORES * NUM_SUBCORES
BPW = BATCH // NW
LANES = 16
ETILES = EMBED_DIM // 8
BTILES = BPW // 128
NPAIR = FIELDS // 2

_mesh = plsc.VectorSubcoreMesh(core_axis_name="c", subcore_axis_name="s")


@functools.partial(
    pl.kernel,
    mesh=_mesh,
    out_type=jax.ShapeDtypeStruct(
        (FIELDS, ETILES, BATCH // 128, 8, 128), jnp.float32),
    scratch_types=[
        pltpu.VMEM((BPW,), jnp.int32),
        pltpu.VMEM((BPW,), jnp.int32),
        pltpu.VMEM((BPW, EMBED_DIM), jnp.float32),
        pltpu.VMEM((BPW, EMBED_DIM), jnp.float32),
        pltpu.VMEM((ETILES, BTILES, 8, 128), jnp.float32),
        pltpu.VMEM((ETILES, BTILES, 8, 128), jnp.float32),
        pltpu.SemaphoreType.DMA,
        pltpu.SemaphoreType.DMA,
        pltpu.SemaphoreType.DMA,
    ],
    compiler_params=pltpu.CompilerParams(
        use_tc_tiling_on_sc=False, needs_layout_passes=False),
)
def _emb_lookup(idx_hbm, table_hbm, out_hbm,
                idx_a, idx_b, rows_a, rows_b, t_a, t_b,
                sem_i, sem_g, sem_o):
    wid = lax.axis_index("s") * NUM_CORES + lax.axis_index("c")
    base_b = wid * BPW
    bt0 = wid * BTILES
    ar16 = jnp.arange(LANES, dtype=jnp.int32)

    def idx_src(f):
        return idx_hbm.at[pl.ds(f * BATCH + base_b, BPW)]

    def transpose_into(rows_v, t_v):
        def g_body(g, c):
            q = g // 8
            j = g % 8
            row_ids = g * LANES + ar16
            vals = [
                plsc.load_gather(
                    rows_v, [row_ids, jnp.full((LANES,), e, jnp.int32)])
                for e in range(EMBED_DIM)
            ]
            for e in range(EMBED_DIM):
                t_v[e // 8, q, e % 8, pl.ds(j * LANES, LANES)] = (
                    jnp.abs(vals[e]))
            return c
        lax.fori_loop(0, BPW // LANES, g_body, 0)

    def fire_out(t_v, f):
        for e_t in range(ETILES):
            pltpu.async_copy(t_v.at[e_t],
                             out_hbm.at[f, e_t, pl.ds(bt0, BTILES)], sem_o)

    def drain_out():
        for e_t in range(ETILES):
            pltpu.make_async_copy(
                t_a.at[e_t], out_hbm.at[0, e_t, pl.ds(bt0, BTILES)],
                sem_o).wait()

    def drain_gather(rows_v):
        pltpu.make_async_copy(
            table_hbm.at[pl.ds(0, BPW)], rows_v, sem_g).wait()

    def drain_idx(idx_v):
        pltpu.make_async_copy(idx_src(0), idx_v, sem_i).wait()

    pltpu.sync_copy(idx_src(0), idx_a)
    pltpu.async_copy(table_hbm.at[idx_a], rows_a, sem_g)
    pltpu.async_copy(idx_src(1), idx_b, sem_i)

    def pair_body(k, carry):
        f0 = 2 * k
        f1 = f0 + 1

        drain_gather(rows_a)

        drain_idx(idx_b)
        pltpu.async_copy(table_hbm.at[idx_b], rows_b, sem_g)

        @pl.when(f0 + 2 < FIELDS)
        def _():
            pltpu.async_copy(idx_src(f0 + 2), idx_a, sem_i)

        @pl.when(k >= 1)
        def _():
            drain_out()

        transpose_into(rows_a, t_a)
        fire_out(t_a, f0)

        drain_gather(rows_b)

        @pl.when(f1 + 1 < FIELDS)
        def _():
            drain_idx(idx_a)
            pltpu.async_copy(table_hbm.at[idx_a], rows_a, sem_g)

        @pl.when(f1 + 2 < FIELDS)
        def _():
            pltpu.async_copy(idx_src(f1 + 2), idx_b, sem_i)

        @pl.when(k >= 1)
        def _():
            drain_out()

        transpose_into(rows_b, t_b)
        fire_out(t_b, f1)
        return carry

    lax.fori_loop(0, NPAIR, pair_body, 0)

    drain_out()
    drain_out()


def kernel(inputs, table):
    idx_t = inputs.T.reshape(-1).astype(jnp.int32)
    out5 = _emb_lookup(idx_t, table)
    return out5.transpose((2, 4, 0, 1, 3)).reshape(BATCH, FIELDS, EMBED_DIM)
